# baseline (speedup 1.0000x reference)
import functools

import jax
import jax.numpy as jnp
from jax import lax
from jax.experimental import pallas as pl
from jax.experimental.pallas import tpu as pltpu
from jax.experimental.pallas import tpu_sc as plsc

BATCH = 16384
SEQ = 200
EMBED = 128
PAD_IDX = 2
TOTAL = BATCH * SEQ

NC = 2
NS = 16
NW = NC * NS
PER_W = TOTAL // NW
CHUNK = 128
K = 16
NBLK = PER_W // (CHUNK * K)

_mesh = plsc.VectorSubcoreMesh(core_axis_name="c", subcore_axis_name="s")


@functools.partial(
    pl.kernel,
    out_type=jax.ShapeDtypeStruct((TOTAL, EMBED), jnp.float32),
    mesh=_mesh,
    scratch_types=[
        pltpu.VMEM_SHARED((3, EMBED), jnp.float32),
        pltpu.VMEM((K, CHUNK), jnp.int32),
        pltpu.VMEM((CHUNK, EMBED), jnp.float32),
        pltpu.VMEM((CHUNK, EMBED), jnp.float32),
        pltpu.VMEM((CHUNK, EMBED), jnp.float32),
        pltpu.VMEM((CHUNK, EMBED), jnp.float32),
        pltpu.SemaphoreType.DMA,
        pltpu.SemaphoreType.DMA,
        pltpu.SemaphoreType.DMA,
        pltpu.SemaphoreType.DMA,
        pltpu.SemaphoreType.DMA,
        pltpu.SemaphoreType.DMA,
        pltpu.SemaphoreType.DMA,
        pltpu.SemaphoreType.DMA,
    ],
)
def _sc_embed(idx_hbm, table_hbm, out_hbm, table_v, idx_v,
              rows0, rows1, rows2, rows3,
              gs0, gs1, gs2, gs3, ss0, ss1, ss2, ss3):
    wid = lax.axis_index("s") * NC + lax.axis_index("c")
    crow0 = wid * (PER_W // CHUNK)
    rbase = wid * PER_W
    bufs = (rows0, rows1, rows2, rows3)
    gsems = (gs0, gs1, gs2, gs3)
    ssems = (ss0, ss1, ss2, ss3)
    nbuf = 4
    lookahead = 2

    @pl.when(lax.axis_index("s") == 0)
    def _stage_table():
        pltpu.sync_copy(table_hbm, table_v)

    plsc.subcore_barrier()

    def block(blk, carry):
        pltpu.sync_copy(idx_hbm.at[pl.ds(crow0 + blk * K, K)], idx_v)
        gathers = [None] * nbuf
        stores = [None] * nbuf

        def start_store(j):
            b = j % nbuf
            gathers[b].wait()
            stores[b] = pltpu.async_copy(
                bufs[b],
                out_hbm.at[pl.ds(rbase + (blk * K + j) * CHUNK, CHUNK)],
                ssems[b],
            )

        for j in range(K):
            b = j % nbuf
            if stores[b] is not None:
                stores[b].wait()
            gathers[b] = pltpu.async_copy(
                table_v.at[idx_v.at[j]], bufs[b], gsems[b])
            if j >= lookahead:
                start_store(j - lookahead)
        for j in range(K - lookahead, K):
            start_store(j)
        for b in range(nbuf):
            stores[b].wait()
        return carry

    lax.fori_loop(0, NBLK, block, 0)


def kernel(inputs, table):
    pad_mask = (jnp.arange(3) != PAD_IDX).astype(table.dtype)[:, None]
    masked_table = table * pad_mask
    idx2d = inputs.reshape(TOTAL // CHUNK, CHUNK)
    out = _sc_embed(idx2d, masked_table)
    return out.reshape(BATCH, SEQ, EMBED)


# speedup vs baseline: 1.0316x; 1.0316x over previous
import functools

import jax
import jax.numpy as jnp
from jax import lax
from jax.experimental import pallas as pl
from jax.experimental.pallas import tpu as pltpu
from jax.experimental.pallas import tpu_sc as plsc

BATCH = 16384
SEQ = 200
EMBED = 128
PAD_IDX = 2
TOTAL = BATCH * SEQ

NC = 2
NS = 16
NW = NC * NS
PER_W = TOTAL // NW
CHUNK = 128
K = 16
NBLK = PER_W // (CHUNK * K)

_mesh = plsc.VectorSubcoreMesh(core_axis_name="c", subcore_axis_name="s")


@functools.partial(
    pl.kernel,
    out_type=jax.ShapeDtypeStruct((TOTAL, EMBED), jnp.float32),
    mesh=_mesh,
    scratch_types=[
        pltpu.VMEM_SHARED((3, EMBED), jnp.float32),
        pltpu.VMEM((K, CHUNK), jnp.int32),
        pltpu.VMEM((CHUNK, EMBED), jnp.float32),
        pltpu.VMEM((CHUNK, EMBED), jnp.float32),
        pltpu.SemaphoreType.DMA,
        pltpu.SemaphoreType.DMA,
        pltpu.SemaphoreType.DMA,
        pltpu.SemaphoreType.DMA,
    ],
)
def _sc_embed(idx_hbm, table_hbm, out_hbm, table_v, idx_v, rows0, rows1,
              gs0, gs1, ss0, ss1):
    wid = lax.axis_index("s") * NC + lax.axis_index("c")
    crow0 = wid * (PER_W // CHUNK)
    rbase = wid * PER_W
    bufs = (rows0, rows1)
    gsems = (gs0, gs1)
    ssems = (ss0, ss1)

    @pl.when(lax.axis_index("s") == 0)
    def _stage_table():
        pltpu.sync_copy(table_hbm, table_v)

    plsc.subcore_barrier()

    def block(blk, carry):
        pltpu.sync_copy(idx_hbm.at[pl.ds(crow0 + blk * K, K)], idx_v)
        gathers = [None, None]
        stores = [None, None]
        for j in range(K):
            b = j & 1
            if stores[b] is not None:
                stores[b].wait()
            gathers[b] = pltpu.async_copy(
                table_v.at[idx_v.at[j]], bufs[b], gsems[b])
            pb = (j - 1) & 1
            if j >= 1:
                gathers[pb].wait()
                stores[pb] = pltpu.async_copy(
                    bufs[pb],
                    out_hbm.at[pl.ds(rbase + (blk * K + j - 1) * CHUNK, CHUNK)],
                    ssems[pb],
                )
        lb = (K - 1) & 1
        gathers[lb].wait()
        stores[lb] = pltpu.async_copy(
            bufs[lb],
            out_hbm.at[pl.ds(rbase + (blk * K + K - 1) * CHUNK, CHUNK)],
            ssems[lb],
        )
        stores[0].wait()
        stores[1].wait()
        return carry

    lax.fori_loop(0, NBLK, block, 0)


def kernel(inputs, table):
    pad_mask = (jnp.arange(3) != PAD_IDX).astype(table.dtype)[:, None]
    masked_table = table * pad_mask
    idx2d = inputs.reshape(TOTAL // CHUNK, CHUNK)
    out = _sc_embed(idx2d, masked_table)
    return out.reshape(BATCH, SEQ, EMBED)
